# Initial kernel scaffold; baseline (speedup 1.0000x reference)
import jax
import jax.numpy as jnp
from jax.experimental import pallas as pl


def kernel(x, edge_index, params):
    raise NotImplementedError("write your pallas kernel here")


# speedup vs baseline: 1.2714x; 1.2714x over previous
import functools

import jax
import jax.numpy as jnp
from jax import lax
from jax.experimental import pallas as pl
from jax.experimental.pallas import tpu as pltpu
from jax.experimental.pallas import tpu_sc as plsc

N = 10000
E = 320000
H = 128
HEADS = 4
EPS = 1e-5

N1, N1P = 5000, 5120
N2, N2P = 2500, 2560
EP = 327680
ACC_ROWS = 10112

NCORES, NSUB = 2, 16

f32 = jnp.float32
i32 = jnp.int32


def _k1_body(x_ref, w_ref, b_ref, h_ref, h1_ref, h2_ref):
    h = jnp.dot(x_ref[...], w_ref[...], preferred_element_type=f32) + b_ref[...]
    h_ref[...] = h
    r = h.reshape(h.shape[0] // 2, 2, H)
    h1_ref[...] = (r[:, 0, :] + r[:, 1, :]) * 0.5
    r4 = h.reshape(h.shape[0] // 4, 4, H)
    h2_ref[...] = (r4[:, 0, :] + r4[:, 1, :] + r4[:, 2, :] + r4[:, 3, :]) * 0.25


def _k1(x, w, b):
    B = 2048
    grid = (pl.cdiv(N, B),)
    return pl.pallas_call(
        _k1_body,
        grid=grid,
        in_specs=[
            pl.BlockSpec((B, H), lambda i: (i, 0)),
            pl.BlockSpec((H, H), lambda i: (0, 0)),
            pl.BlockSpec((1, H), lambda i: (0, 0)),
        ],
        out_specs=[
            pl.BlockSpec((B, H), lambda i: (i, 0)),
            pl.BlockSpec((B // 2, H), lambda i: (i, 0)),
            pl.BlockSpec((B // 4, H), lambda i: (i, 0)),
        ],
        out_shape=[
            jax.ShapeDtypeStruct((N, H), f32),
            jax.ShapeDtypeStruct((N // 2, H), f32),
            jax.ShapeDtypeStruct((N // 4, H), f32),
        ],
    )(x, w, b)


TRASH1 = (0 * 2560 * N1P + (N1P - 1), 1 * 2560 * N1P + (N1P - 1))
TRASH2 = (0 * 1280 * N2P + (N2P - 1), 1 * 1280 * N2P + (N2P - 1))


def _k2a_body(ei_ref, o1_ref, o2_ref):
    s = ei_ref[0]
    d = ei_ref[1]
    c1s = s // 2
    c1d = d // 2
    v1 = c1s != c1d
    f1 = c1d * N1P + c1s
    b1 = c1s * N1P + c1d
    fhi1 = c1d >= 2560
    bhi1 = c1s >= 2560
    c2s = s // 4
    c2d = d // 4
    v2 = c2s != c2d
    f2 = c2d * N2P + c2s
    b2 = c2s * N2P + c2d
    fhi2 = c2d >= 1280
    bhi2 = c2s >= 1280
    for c in range(2):
        hi = c == 1
        o1_ref[c, 0] = jnp.where(v1 & (fhi1 == hi), f1, TRASH1[c])
        o1_ref[c, 1] = jnp.where(v1 & (bhi1 == hi), b1, TRASH1[c])
        o2_ref[c, 0] = jnp.where(v2 & (fhi2 == hi), f2, TRASH2[c])
        o2_ref[c, 1] = jnp.where(v2 & (bhi2 == hi), b2, TRASH2[c])


def _k2a(ei3):
    return pl.pallas_call(
        _k2a_body,
        out_shape=[
            jax.ShapeDtypeStruct((2, 2, 2500, 128), i32),
            jax.ShapeDtypeStruct((2, 2, 2500, 128), i32),
        ],
    )(ei3)


ZWORDS = 51200


def _k2b_body(idx1, idx2, zhbm, ohbm, a1, a2, zbuf, ibuf, ones_v, sem):
    c = lax.axis_index("c")
    s = lax.axis_index("s")
    pltpu.sync_copy(zhbm, zbuf)
    base1 = (c * 2560 + s * 160) * N1P
    for j in range(16):
        pltpu.sync_copy(zbuf, a1.at[pl.ds(base1 + j * ZWORDS, ZWORDS)])
    base2 = (c * 1280 + s * 80) * N2P
    for j in range(4):
        pltpu.sync_copy(zbuf, a2.at[pl.ds(base2 + j * ZWORDS, ZWORDS)])
    plsc.subcore_barrier()
    pltpu.sync_copy(ohbm, ones_v)

    def scatter_list(idx, dst, jj):
        pltpu.sync_copy(idx.at[c, pl.ds(s * 320 + jj * 16, 16), :], ibuf)
        descs = [
            pltpu.async_copy(ones_v, dst.at[ibuf.at[j]], sem) for j in range(16)
        ]
        for dsc in descs:
            dsc.wait()

    for jj in range(20):
        scatter_list(idx1, a1, jj)
    for jj in range(20):
        scatter_list(idx2, a2, jj)


def _k2b(idx1, idx2):
    zhbm = jnp.zeros((ZWORDS,), f32)
    ohbm = jnp.ones((128,), f32)
    mesh = plsc.VectorSubcoreMesh(core_axis_name="c", subcore_axis_name="s")
    return pl.kernel(
        _k2b_body,
        out_type=[
            jax.ShapeDtypeStruct((N1P * N1P,), f32),
            jax.ShapeDtypeStruct((N2P * N2P,), f32),
        ],
        mesh=mesh,
        scratch_types=[
            pltpu.VMEM((ZWORDS,), f32),
            pltpu.VMEM((16, 128), i32),
            pltpu.VMEM((128,), f32),
            pltpu.SemaphoreType.DMA,
        ],
    )(idx1, idx2, zhbm, ohbm)


def _k3_body(h, src2d, dst2d, zrows, out, acc, sidx, didx, rows, gsem, ssem):
    c = lax.axis_index("c")
    s = lax.axis_index("s")
    pltpu.sync_copy(zrows, acc.at[pl.ds(s * 632, 632), :])
    plsc.subcore_barrier()
    r0 = (c * NSUB + s) * 80
    for jj in range(10):
        pltpu.sync_copy(src2d.at[pl.ds(r0 + jj * 8, 8), :], sidx)
        pltpu.sync_copy(dst2d.at[pl.ds(r0 + jj * 8, 8), :], didx)
        for w in range(4):
            g = [
                pltpu.async_copy(h.at[sidx.at[w * 2 + k]], rows.at[k], gsem)
                for k in range(2)
            ]
            for dsc in g:
                dsc.wait()
            sc = [
                pltpu.async_copy(
                    rows.at[k], acc.at[didx.at[w * 2 + k]], ssem, add=True
                )
                for k in range(2)
            ]
            for dsc in sc:
                dsc.wait()
    plsc.subcore_barrier()
    pltpu.sync_copy(
        acc.at[pl.ds(s * 624, 624), :], out.at[c, pl.ds(s * 624, 624), :]
    )

    @pl.when(s == 0)
    def _():
        pltpu.sync_copy(
            acc.at[pl.ds(9984, 16), :], out.at[c, pl.ds(9984, 16), :]
        )


def _k3(h, src2d, dst2d):
    zrows = jnp.zeros((632, H), f32)
    mesh = plsc.VectorSubcoreMesh(core_axis_name="c", subcore_axis_name="s")
    return pl.kernel(
        _k3_body,
        out_type=jax.ShapeDtypeStruct((2, N, H), f32),
        mesh=mesh,
        scratch_types=[
            pltpu.VMEM_SHARED((ACC_ROWS, H), f32),
            pltpu.VMEM((8, 128), i32),
            pltpu.VMEM((8, 128), i32),
            pltpu.VMEM((2, 128, H), f32),
            pltpu.SemaphoreType.DMA,
            pltpu.SemaphoreType.DMA,
        ],
    )(h, src2d, dst2d, zrows)


def _k4_body(p0_ref, p1_ref, h_ref, wrel_ref, wroot_ref, b_ref, o_ref):
    agg = p0_ref[0] + p1_ref[0]
    o = (
        jnp.dot(agg, wrel_ref[...], preferred_element_type=f32)
        + jnp.dot(h_ref[...], wroot_ref[...], preferred_element_type=f32)
        + b_ref[...]
    )
    o_ref[...] = jnp.maximum(o, 0.0)


def _k4(parts, h, wrel, wroot, b):
    B = 2000
    return pl.pallas_call(
        _k4_body,
        grid=(N // B,),
        in_specs=[
            pl.BlockSpec((1, B, H), lambda i: (0, i, 0)),
            pl.BlockSpec((1, B, H), lambda i: (1, i, 0)),
            pl.BlockSpec((B, H), lambda i: (i, 0)),
            pl.BlockSpec((H, H), lambda i: (0, 0)),
            pl.BlockSpec((H, H), lambda i: (0, 0)),
            pl.BlockSpec((1, H), lambda i: (0, 0)),
        ],
        out_specs=pl.BlockSpec((B, H), lambda i: (i, 0)),
        out_shape=jax.ShapeDtypeStruct((N, H), f32),
    )(parts, parts, h, wrel, wroot, b)


def _k5_body(a_ref, hfull_ref, hblk_ref, wl_ref, wr_ref, b_ref, o_ref):
    a = jnp.minimum(a_ref[...], 1.0)
    col = lax.broadcasted_iota(i32, a.shape, 1)
    a = jnp.where(col < N1, a, 0.0)
    s = jnp.dot(a, hfull_ref[...], preferred_element_type=f32)
    cnt = jnp.sum(a, axis=1, keepdims=True)
    mean = s / jnp.maximum(cnt, 1.0)
    o = (
        jnp.dot(mean, wl_ref[...], preferred_element_type=f32)
        + jnp.dot(hblk_ref[...], wr_ref[...], preferred_element_type=f32)
        + b_ref[...]
    )
    o_ref[...] = jnp.maximum(o, 0.0)


def _k5(a1, h1, wl, wr, b):
    B = 512
    return pl.pallas_call(
        _k5_body,
        grid=(N1P // B,),
        in_specs=[
            pl.BlockSpec((B, N1P), lambda i: (i, 0)),
            pl.BlockSpec((N1P, H), lambda i: (0, 0)),
            pl.BlockSpec((B, H), lambda i: (i, 0)),
            pl.BlockSpec((H, H), lambda i: (0, 0)),
            pl.BlockSpec((H, H), lambda i: (0, 0)),
            pl.BlockSpec((1, H), lambda i: (0, 0)),
        ],
        out_specs=pl.BlockSpec((B, H), lambda i: (i, 0)),
        out_shape=jax.ShapeDtypeStruct((N1P, H), f32),
    )(a1, h1, h1, wl, wr, b)


def _k6_body(a_ref, hfull_ref, hblk_ref, w_ref, asrc_ref, adst_ref, b_ref, o_ref):
    mask = a_ref[...] >= 0.5
    col = lax.broadcasted_iota(i32, mask.shape, 1)
    mask = mask & (col < N2)
    hfull = hfull_ref[...]
    hblk = hblk_ref[...]
    acc = jnp.zeros((hblk.shape[0], H), f32)
    for hd in range(HEADS):
        wh = w_ref[:, hd * H:(hd + 1) * H]
        xh = jnp.dot(hfull, wh, preferred_element_type=f32)
        xh_blk = jnp.dot(hblk, wh, preferred_element_type=f32)
        a_s = lax.dot_general(
            asrc_ref[hd][None, :], xh, (((1,), (1,)), ((), ())),
            preferred_element_type=f32)
        a_d = jnp.dot(xh_blk, adst_ref[hd][:, None],
                      preferred_element_type=f32)
        e = a_s + a_d
        e = jnp.where(e >= 0.0, e, 0.2 * e)
        m = jnp.max(jnp.where(mask, e, -1e30), axis=1, keepdims=True)
        m = jnp.where(m > -1e29, m, 0.0)
        p = jnp.where(mask, jnp.exp(e - m), 0.0)
        z = jnp.sum(p, axis=1, keepdims=True)
        num = jnp.dot(p, xh, preferred_element_type=f32)
        acc = acc + num / (z + 1e-16)
    o_ref[...] = jnp.maximum(acc * (1.0 / HEADS) + b_ref[...], 0.0)


def _k6(a2, h2, w, asrc, adst, b):
    B = 512
    return pl.pallas_call(
        _k6_body,
        grid=(N2P // B,),
        in_specs=[
            pl.BlockSpec((B, N2P), lambda i: (i, 0)),
            pl.BlockSpec((N2P, H), lambda i: (0, 0)),
            pl.BlockSpec((B, H), lambda i: (i, 0)),
            pl.BlockSpec((H, HEADS * H), lambda i: (0, 0)),
            pl.BlockSpec((HEADS, H), lambda i: (0, 0)),
            pl.BlockSpec((HEADS, H), lambda i: (0, 0)),
            pl.BlockSpec((1, H), lambda i: (0, 0)),
        ],
        out_specs=pl.BlockSpec((B, H), lambda i: (i, 0)),
        out_shape=jax.ShapeDtypeStruct((N2P, H), f32),
    )(a2, h2, h2, w, asrc, adst, b)


def _k7_body(h0_ref, h1_ref, h2_ref, g_ref, bn_ref, wf_ref, bf_ref, o_ref):
    B = h0_ref.shape[0]
    h1 = h1_ref[...]
    h2 = h2_ref[...]
    h1r = jnp.broadcast_to(h1[:, None, :], (B // 2, 2, H)).reshape(B, H)
    h2r = jnp.broadcast_to(h2[:, None, :], (B // 4, 4, H)).reshape(B, H)
    comb = (h0_ref[...] + h1r + h2r) * (1.0 / 3.0)
    mu = jnp.mean(comb, axis=1, keepdims=True)
    dc = comb - mu
    var = jnp.mean(dc * dc, axis=1, keepdims=True)
    normed = dc * lax.rsqrt(var + EPS) * g_ref[...] + bn_ref[...]
    o_ref[...] = (
        jnp.dot(normed, wf_ref[...], preferred_element_type=f32) + bf_ref[...]
    )


def _k7(h0, h1, h2, g, bn, wf, bf):
    B = 2048
    return pl.pallas_call(
        _k7_body,
        grid=(pl.cdiv(N, B),),
        in_specs=[
            pl.BlockSpec((B, H), lambda i: (i, 0)),
            pl.BlockSpec((B // 2, H), lambda i: (i, 0)),
            pl.BlockSpec((B // 4, H), lambda i: (i, 0)),
            pl.BlockSpec((1, H), lambda i: (0, 0)),
            pl.BlockSpec((1, H), lambda i: (0, 0)),
            pl.BlockSpec((H, H), lambda i: (0, 0)),
            pl.BlockSpec((1, H), lambda i: (0, 0)),
        ],
        out_specs=pl.BlockSpec((B, H), lambda i: (i, 0)),
        out_shape=jax.ShapeDtypeStruct((N, H), f32),
    )(h0, h1, h2, g, bn, wf, bf)


def kernel(x, edge_index, params):
    p = params

    h, h1_0, h2_0 = _k1(x, p['node_proj_w'], p['node_proj_b'][None, :])

    ei3 = edge_index.reshape(2, 2500, 128)
    idx1, idx2 = _k2a(ei3)
    pad1 = jnp.stack(
        [jnp.full((120, 128), TRASH1[0], i32), jnp.full((120, 128), TRASH1[1], i32)]
    )
    pad2 = jnp.stack(
        [jnp.full((120, 128), TRASH2[0], i32), jnp.full((120, 128), TRASH2[1], i32)]
    )
    idx1 = jnp.concatenate([idx1.reshape(2, 5000, 128), pad1], axis=1)
    idx2 = jnp.concatenate([idx2.reshape(2, 5000, 128), pad2], axis=1)
    a1f, a2f = _k2b(idx1, idx2)
    a1 = a1f.reshape(N1P, N1P)
    a2 = a2f.reshape(N2P, N2P)

    epad = jnp.zeros((2, EP - E), i32).at[1].set(N)
    eip = jnp.concatenate([edge_index, epad], axis=1)
    src2d = eip[0].reshape(2560, 128)
    dst2d = eip[1].reshape(2560, 128)
    h0 = h
    for i in range(2):
        parts = _k3(h0, src2d, dst2d)
        h0 = _k4(parts, h0, p['gc%d_w_rel' % i], p['gc%d_w_root' % i],
                 p['gc%d_b' % i][None, :])

    h1 = jnp.concatenate([h1_0, jnp.zeros((N1P - N1, H), f32)], axis=0)
    for i in range(2):
        h1 = _k5(a1, h1, p['sage%d_w_l' % i], p['sage%d_w_r' % i],
                 p['sage%d_b' % i][None, :])

    h2 = jnp.concatenate([h2_0, jnp.zeros((N2P - N2, H), f32)], axis=0)
    for i in range(2):
        h2 = _k6(a2, h2, p['gat%d_w' % i], p['gat%d_att_src' % i],
                 p['gat%d_att_dst' % i], p['gat%d_b' % i][None, :])

    return _k7(h0, h1[:N1], h2[:N2], p['final_norm_g'][None, :],
               p['final_norm_b'][None, :], p['final_proj_w'],
               p['final_proj_b'][None, :])


# speedup vs baseline: 30.2869x; 23.8225x over previous
import functools

import jax
import jax.numpy as jnp
from jax import lax
from jax.experimental import pallas as pl
from jax.experimental.pallas import tpu as pltpu
from jax.experimental.pallas import tpu_sc as plsc

N = 10000
E = 320000
H = 128
HEADS = 4
EPS = 1e-5

N1, N1P = 5000, 5120
N2, N2P = 2500, 2560
EP = 327680
ACC_ROWS = 10112

NCORES, NSUB = 2, 16

f32 = jnp.float32
i32 = jnp.int32


def _k1_body(x_ref, w_ref, b_ref, h_ref, h1_ref, h2_ref):
    h = jnp.dot(x_ref[...], w_ref[...], preferred_element_type=f32) + b_ref[...]
    h_ref[...] = h
    r = h.reshape(h.shape[0] // 2, 2, H)
    h1_ref[...] = (r[:, 0, :] + r[:, 1, :]) * 0.5
    r4 = h.reshape(h.shape[0] // 4, 4, H)
    h2_ref[...] = (r4[:, 0, :] + r4[:, 1, :] + r4[:, 2, :] + r4[:, 3, :]) * 0.25


def _k1(x, w, b):
    B = 2048
    grid = (pl.cdiv(N, B),)
    return pl.pallas_call(
        _k1_body,
        grid=grid,
        in_specs=[
            pl.BlockSpec((B, H), lambda i: (i, 0)),
            pl.BlockSpec((H, H), lambda i: (0, 0)),
            pl.BlockSpec((1, H), lambda i: (0, 0)),
        ],
        out_specs=[
            pl.BlockSpec((B, H), lambda i: (i, 0)),
            pl.BlockSpec((B // 2, H), lambda i: (i, 0)),
            pl.BlockSpec((B // 4, H), lambda i: (i, 0)),
        ],
        out_shape=[
            jax.ShapeDtypeStruct((N, H), f32),
            jax.ShapeDtypeStruct((N // 2, H), f32),
            jax.ShapeDtypeStruct((N // 4, H), f32),
        ],
    )(x, w, b)


i16 = jnp.int16
PAN1 = 640
PAN_ELEMS = 640 * N1P


def _k2a_body(ei_ref, o1_ref, o2_ref, v1_ref, v2_ref):
    s = ei_ref[0]
    d = ei_ref[1]
    c1s = s // 2
    c1d = d // 2
    v1 = c1s != c1d
    c2s = s // 4
    c2d = d // 4
    v2 = c2s != c2d
    for c in range(2):
        for p in range(4):
            base = c * 2560 + p * PAN1
            infw = v1 & (c1d >= base) & (c1d < base + PAN1)
            inbw = v1 & (c1s >= base) & (c1s < base + PAN1)
            lf = jnp.where(infw, (c1d - base) * N1P + c1s, N1P - 1)
            lb = jnp.where(inbw, (c1s - base) * N1P + c1d, N1P - 1)
            o1_ref[c, p, 0] = lf >> 1
            o1_ref[c, p, 1] = lb >> 1
            v1_ref[c, p, 0] = jnp.int32(1) << (16 * (lf & 1))
            v1_ref[c, p, 1] = jnp.int32(1) << (16 * (lb & 1))
        base2 = c * 1280
        infw2 = v2 & (c2d >= base2) & (c2d < base2 + 1280)
        inbw2 = v2 & (c2s >= base2) & (c2s < base2 + 1280)
        lf2 = jnp.where(infw2, (c2d - base2) * N2P + c2s, N2P - 1)
        lb2 = jnp.where(inbw2, (c2s - base2) * N2P + c2d, N2P - 1)
        o2_ref[c, 0] = lf2 >> 1
        o2_ref[c, 1] = lb2 >> 1
        v2_ref[c, 0] = jnp.int32(1) << (16 * (lf2 & 1))
        v2_ref[c, 1] = jnp.int32(1) << (16 * (lb2 & 1))


def _k2a(ei3):
    B = 512
    return pl.pallas_call(
        _k2a_body,
        grid=(pl.cdiv(2500, B),),
        in_specs=[pl.BlockSpec((2, B, 128), lambda i: (0, i, 0))],
        out_specs=[
            pl.BlockSpec((2, 4, 2, B, 128), lambda i: (0, 0, 0, i, 0)),
            pl.BlockSpec((2, 2, B, 128), lambda i: (0, 0, i, 0)),
            pl.BlockSpec((2, 4, 2, B, 128), lambda i: (0, 0, 0, i, 0)),
            pl.BlockSpec((2, 2, B, 128), lambda i: (0, 0, i, 0)),
        ],
        out_shape=[
            jax.ShapeDtypeStruct((2, 4, 2, 2500, 128), i32),
            jax.ShapeDtypeStruct((2, 2, 2500, 128), i32),
            jax.ShapeDtypeStruct((2, 4, 2, 2500, 128), i32),
            jax.ShapeDtypeStruct((2, 2, 2500, 128), i32),
        ],
    )(ei3)


PAN_WORDS = PAN_ELEMS // 2
ZSTRIPE = PAN_WORDS // NSUB


CHUNK = 12800


def _k2b_scatter_pass(idx, val, s, pan, ibuf, vbuf, tmp, zpan, sem, out,
                      out_base):
    poff = pl.multiple_of(s * ZSTRIPE, 8)
    ooff = pl.multiple_of(out_base + s * ZSTRIPE, 8)
    pltpu.sync_copy(zpan, tmp)
    for j in range(8):
        pltpu.sync_copy(tmp, pan.at[pl.ds(poff + j * CHUNK, CHUNK)])
    plsc.subcore_barrier()

    r0 = s * 320
    prev = None
    for b in range(20):
        buf = b % 2
        pltpu.sync_copy(idx.at[pl.ds(r0 + b * 16, 16), :], ibuf.at[buf])
        pltpu.sync_copy(val.at[pl.ds(r0 + b * 16, 16), :], vbuf.at[buf])
        cur = [
            pltpu.async_copy(
                vbuf.at[buf, j], pan.at[ibuf.at[buf, j]], sem, add=True)
            for j in range(16)
        ]
        if prev is not None:
            for dsc in prev:
                dsc.wait()
        prev = cur
    for dsc in prev:
        dsc.wait()
    plsc.subcore_barrier()
    for j in range(8):
        pltpu.sync_copy(pan.at[pl.ds(poff + j * CHUNK, CHUNK)], tmp)
        pltpu.sync_copy(tmp, out.at[pl.ds(ooff + j * CHUNK, CHUNK)])
    plsc.subcore_barrier()


def _make_k2b(npanels):
    def body(idx, val, zpan, out, pan, ibuf, vbuf, tmp, sem):
        c = lax.axis_index("c")
        s = lax.axis_index("s")
        for p in range(npanels):
            _k2b_scatter_pass(
                idx.at[c, p], val.at[c, p], s, pan, ibuf, vbuf, tmp, zpan,
                sem, out, (c * npanels + p) * PAN_WORDS)

    mesh = plsc.VectorSubcoreMesh(core_axis_name="c", subcore_axis_name="s")
    return pl.kernel(
        body,
        out_type=jax.ShapeDtypeStruct((2 * npanels * PAN_WORDS,), i32),
        mesh=mesh,
        scratch_types=[
            pltpu.VMEM_SHARED((PAN_WORDS,), i32),
            pltpu.VMEM((2, 16, 128), i32),
            pltpu.VMEM((2, 16, 128), i32),
            pltpu.VMEM((CHUNK,), i32),
            pltpu.SemaphoreType.DMA,
        ],
    )


def _k2b(idx1, idx2, val1, val2):
    zpan = jnp.zeros((CHUNK,), i32)
    run2 = _make_k2b(2)
    run1 = _make_k2b(1)
    oa = run2(idx1[:, :2], val1[:, :2], zpan).reshape(2, 2 * PAN_WORDS)
    ob = run2(idx1[:, 2:], val1[:, 2:], zpan).reshape(2, 2 * PAN_WORDS)
    oc = run1(idx2[:, None], val2[:, None], zpan).reshape(2, PAN_WORDS)
    a1w = jnp.concatenate([oa[0], ob[0], oa[1], ob[1]])
    a2w = jnp.concatenate([oc[0], oc[1]])
    return a1w, a2w


def _k3_body(h, src2d, dst2d, zrows, out, acc, sidx, didx, rows, gsem, ssem):
    c = lax.axis_index("c")
    s = lax.axis_index("s")
    pltpu.sync_copy(zrows, acc.at[pl.ds(s * 632, 632), :])
    plsc.subcore_barrier()
    r0 = (c * NSUB + s) * 80
    for jj in range(10):
        pltpu.sync_copy(src2d.at[pl.ds(r0 + jj * 8, 8), :], sidx)
        pltpu.sync_copy(dst2d.at[pl.ds(r0 + jj * 8, 8), :], didx)
        for w in range(4):
            g = [
                pltpu.async_copy(h.at[sidx.at[w * 2 + k]], rows.at[k], gsem)
                for k in range(2)
            ]
            for dsc in g:
                dsc.wait()
            sc = [
                pltpu.async_copy(
                    rows.at[k], acc.at[didx.at[w * 2 + k]], ssem, add=True
                )
                for k in range(2)
            ]
            for dsc in sc:
                dsc.wait()
    plsc.subcore_barrier()
    pltpu.sync_copy(
        acc.at[pl.ds(s * 624, 624), :], out.at[c, pl.ds(s * 624, 624), :]
    )

    @pl.when(s == 0)
    def _():
        pltpu.sync_copy(
            acc.at[pl.ds(9984, 16), :], out.at[c, pl.ds(9984, 16), :]
        )


def _k3(h, src2d, dst2d):
    zrows = jnp.zeros((632, H), f32)
    mesh = plsc.VectorSubcoreMesh(core_axis_name="c", subcore_axis_name="s")
    return pl.kernel(
        _k3_body,
        out_type=jax.ShapeDtypeStruct((2, N, H), f32),
        mesh=mesh,
        scratch_types=[
            pltpu.VMEM_SHARED((ACC_ROWS, H), f32),
            pltpu.VMEM((8, 128), i32),
            pltpu.VMEM((8, 128), i32),
            pltpu.VMEM((2, 128, H), f32),
            pltpu.SemaphoreType.DMA,
            pltpu.SemaphoreType.DMA,
        ],
    )(h, src2d, dst2d, zrows)


def _k4_body(p0_ref, p1_ref, h_ref, wrel_ref, wroot_ref, b_ref, o_ref):
    agg = p0_ref[0] + p1_ref[0]
    o = (
        jnp.dot(agg, wrel_ref[...], preferred_element_type=f32)
        + jnp.dot(h_ref[...], wroot_ref[...], preferred_element_type=f32)
        + b_ref[...]
    )
    o_ref[...] = jnp.maximum(o, 0.0)


def _k4(parts, h, wrel, wroot, b):
    B = 2000
    return pl.pallas_call(
        _k4_body,
        grid=(N // B,),
        in_specs=[
            pl.BlockSpec((1, B, H), lambda i: (0, i, 0)),
            pl.BlockSpec((1, B, H), lambda i: (1, i, 0)),
            pl.BlockSpec((B, H), lambda i: (i, 0)),
            pl.BlockSpec((H, H), lambda i: (0, 0)),
            pl.BlockSpec((H, H), lambda i: (0, 0)),
            pl.BlockSpec((1, H), lambda i: (0, 0)),
        ],
        out_specs=pl.BlockSpec((B, H), lambda i: (i, 0)),
        out_shape=jax.ShapeDtypeStruct((N, H), f32),
    )(parts, parts, h, wrel, wroot, b)


def _k5_body(a_ref, hfull_ref, hblk_ref, wl_ref, wr_ref, b_ref, o_ref):
    a = jnp.minimum(a_ref[...].astype(f32), 1.0)
    col = lax.broadcasted_iota(i32, a.shape, 1)
    a = jnp.where(col < N1, a, 0.0)
    s = jnp.dot(a, hfull_ref[...], preferred_element_type=f32)
    cnt = jnp.sum(a, axis=1, keepdims=True)
    mean = s / jnp.maximum(cnt, 1.0)
    o = (
        jnp.dot(mean, wl_ref[...], preferred_element_type=f32)
        + jnp.dot(hblk_ref[...], wr_ref[...], preferred_element_type=f32)
        + b_ref[...]
    )
    o_ref[...] = jnp.maximum(o, 0.0)


def _k5(a1, h1, wl, wr, b):
    B = 512
    return pl.pallas_call(
        _k5_body,
        grid=(N1P // B,),
        in_specs=[
            pl.BlockSpec((B, N1P), lambda i: (i, 0)),
            pl.BlockSpec((N1P, H), lambda i: (0, 0)),
            pl.BlockSpec((B, H), lambda i: (i, 0)),
            pl.BlockSpec((H, H), lambda i: (0, 0)),
            pl.BlockSpec((H, H), lambda i: (0, 0)),
            pl.BlockSpec((1, H), lambda i: (0, 0)),
        ],
        out_specs=pl.BlockSpec((B, H), lambda i: (i, 0)),
        out_shape=jax.ShapeDtypeStruct((N1P, H), f32),
    )(a1, h1, h1, wl, wr, b)


def _k6_body(a_ref, hfull_ref, hblk_ref, w_ref, asrc_ref, adst_ref, b_ref, o_ref):
    mask = a_ref[...] > 0
    col = lax.broadcasted_iota(i32, mask.shape, 1)
    mask = mask & (col < N2)
    hfull = hfull_ref[...]
    hblk = hblk_ref[...]
    acc = jnp.zeros((hblk.shape[0], H), f32)
    for hd in range(HEADS):
        wh = w_ref[:, hd * H:(hd + 1) * H]
        xh = jnp.dot(hfull, wh, preferred_element_type=f32)
        xh_blk = jnp.dot(hblk, wh, preferred_element_type=f32)
        a_s = lax.dot_general(
            asrc_ref[hd][None, :], xh, (((1,), (1,)), ((), ())),
            preferred_element_type=f32)
        a_d = jnp.dot(xh_blk, adst_ref[hd][:, None],
                      preferred_element_type=f32)
        e = a_s + a_d
        e = jnp.where(e >= 0.0, e, 0.2 * e)
        m = jnp.max(jnp.where(mask, e, -1e30), axis=1, keepdims=True)
        m = jnp.where(m > -1e29, m, 0.0)
        p = jnp.where(mask, jnp.exp(e - m), 0.0)
        z = jnp.sum(p, axis=1, keepdims=True)
        num = jnp.dot(p, xh, preferred_element_type=f32)
        acc = acc + num / (z + 1e-16)
    o_ref[...] = jnp.maximum(acc * (1.0 / HEADS) + b_ref[...], 0.0)


def _k6(a2, h2, w, asrc, adst, b):
    B = 512
    return pl.pallas_call(
        _k6_body,
        grid=(N2P // B,),
        in_specs=[
            pl.BlockSpec((B, N2P), lambda i: (i, 0)),
            pl.BlockSpec((N2P, H), lambda i: (0, 0)),
            pl.BlockSpec((B, H), lambda i: (i, 0)),
            pl.BlockSpec((H, HEADS * H), lambda i: (0, 0)),
            pl.BlockSpec((HEADS, H), lambda i: (0, 0)),
            pl.BlockSpec((HEADS, H), lambda i: (0, 0)),
            pl.BlockSpec((1, H), lambda i: (0, 0)),
        ],
        out_specs=pl.BlockSpec((B, H), lambda i: (i, 0)),
        out_shape=jax.ShapeDtypeStruct((N2P, H), f32),
    )(a2, h2, h2, w, asrc, adst, b)


def _k7_body(h0_ref, h1_ref, h2_ref, g_ref, bn_ref, wf_ref, bf_ref, o_ref):
    B = h0_ref.shape[0]
    h1 = h1_ref[...]
    h2 = h2_ref[...]
    h1r = jnp.broadcast_to(h1[:, None, :], (B // 2, 2, H)).reshape(B, H)
    h2r = jnp.broadcast_to(h2[:, None, :], (B // 4, 4, H)).reshape(B, H)
    comb = (h0_ref[...] + h1r + h2r) * (1.0 / 3.0)
    mu = jnp.mean(comb, axis=1, keepdims=True)
    dc = comb - mu
    var = jnp.mean(dc * dc, axis=1, keepdims=True)
    normed = dc * lax.rsqrt(var + EPS) * g_ref[...] + bn_ref[...]
    o_ref[...] = (
        jnp.dot(normed, wf_ref[...], preferred_element_type=f32) + bf_ref[...]
    )


def _k7(h0, h1, h2, g, bn, wf, bf):
    B = 2048
    return pl.pallas_call(
        _k7_body,
        grid=(pl.cdiv(N, B),),
        in_specs=[
            pl.BlockSpec((B, H), lambda i: (i, 0)),
            pl.BlockSpec((B // 2, H), lambda i: (i, 0)),
            pl.BlockSpec((B // 4, H), lambda i: (i, 0)),
            pl.BlockSpec((1, H), lambda i: (0, 0)),
            pl.BlockSpec((1, H), lambda i: (0, 0)),
            pl.BlockSpec((H, H), lambda i: (0, 0)),
            pl.BlockSpec((1, H), lambda i: (0, 0)),
        ],
        out_specs=pl.BlockSpec((B, H), lambda i: (i, 0)),
        out_shape=jax.ShapeDtypeStruct((N, H), f32),
    )(h0, h1, h2, g, bn, wf, bf)


def kernel(x, edge_index, params):
    p = params

    h, h1_0, h2_0 = _k1(x, p['node_proj_w'], p['node_proj_b'][None, :])

    ei3 = edge_index.reshape(2, 2500, 128)
    idx1, idx2, val1, val2 = _k2a(ei3)
    ipad1 = jnp.full((2, 4, 120, 128), (N1P - 1) >> 1, i32)
    ipad2 = jnp.full((2, 120, 128), (N2P - 1) >> 1, i32)
    vpad1 = jnp.full((2, 4, 120, 128), 1 << 16, i32)
    vpad2 = jnp.full((2, 120, 128), 1 << 16, i32)
    idx1 = jnp.concatenate([idx1.reshape(2, 4, 5000, 128), ipad1], axis=2)
    idx2 = jnp.concatenate([idx2.reshape(2, 5000, 128), ipad2], axis=1)
    val1 = jnp.concatenate([val1.reshape(2, 4, 5000, 128), vpad1], axis=2)
    val2 = jnp.concatenate([val2.reshape(2, 5000, 128), vpad2], axis=1)
    a1w, a2w = _k2b(idx1, idx2, val1, val2)
    a1 = lax.bitcast_convert_type(a1w, i16).reshape(N1P, N1P)
    a2 = lax.bitcast_convert_type(a2w, i16).reshape(N2P, N2P)

    epad = jnp.zeros((2, EP - E), i32).at[1].set(N)
    eip = jnp.concatenate([edge_index, epad], axis=1)
    src2d = eip[0].reshape(2560, 128)
    dst2d = eip[1].reshape(2560, 128)
    h0 = h
    for i in range(2):
        parts = _k3(h0, src2d, dst2d)
        h0 = _k4(parts, h0, p['gc%d_w_rel' % i], p['gc%d_w_root' % i],
                 p['gc%d_b' % i][None, :])

    h1 = jnp.concatenate([h1_0, jnp.zeros((N1P - N1, H), f32)], axis=0)
    for i in range(2):
        h1 = _k5(a1, h1, p['sage%d_w_l' % i], p['sage%d_w_r' % i],
                 p['sage%d_b' % i][None, :])

    h2 = jnp.concatenate([h2_0, jnp.zeros((N2P - N2, H), f32)], axis=0)
    for i in range(2):
        h2 = _k6(a2, h2, p['gat%d_w' % i], p['gat%d_att_src' % i],
                 p['gat%d_att_dst' % i], p['gat%d_b' % i][None, :])

    return _k7(h0, h1[:N1], h2[:N2], p['final_norm_g'][None, :],
               p['final_norm_b'][None, :], p['final_proj_w'],
               p['final_proj_b'][None, :])
